# baseline (device time: 161114 ns/iter reference)
import jax
import jax.numpy as jnp
from jax import lax
from jax.experimental import pallas as pl
from jax.experimental.pallas import tpu as pltpu

N_DEV = 4
SQ = 1024
SKV = 1024
HQ = 8
DH = 128
D = HQ * DH
BLK = 64
SCALE = 0.08838834764831843

ORDER = (0, 4, 1, 5, 2, 6, 3, 7)


def _body(x_ref, wq_ref, kv_ref, wo_ref, out_ref,
          comm_ref, ctx_ref, send_sems, recv_sems):
    my = lax.axis_index("i")
    right = (my + 1) % N_DEV
    left = (my + 3) % N_DEV

    def mk(h, tgt):
        return pltpu.make_async_remote_copy(
            src_ref=comm_ref.at[h],
            dst_ref=comm_ref.at[h],
            send_sem=send_sems.at[h],
            recv_sem=recv_sems.at[h],
            device_id=(tgt,),
            device_id_type=pl.DeviceIdType.MESH,
        )

    snd = [mk(h, right) if h < HQ // 2 else mk(h, left) for h in range(HQ)]

    @pl.when(my == 0)
    def _():
        comm_ref[...] = kv_ref[...]
        for h in range(HQ):
            snd[h].start()

    q_all = jnp.dot(x_ref[...], wq_ref[...],
                    preferred_element_type=jnp.float32).astype(jnp.bfloat16)

    qb = lax.broadcasted_iota(jnp.int32, (SQ, SKV), 0) // BLK
    kb = lax.broadcasted_iota(jnp.int32, (SQ, SKV), 1) // BLK
    mask = kb <= qb

    for h in ORDER:
        cw = h < HQ // 2

        @pl.when(my != 0)
        def _(h=h):
            snd[h].wait_recv()

        fw = (jnp.logical_or(my == 1, my == 2) if cw
              else jnp.logical_or(my == 3, my == 2))

        @pl.when(fw)
        def _(h=h):
            snd[h].start()

        k = comm_ref[h, 0]
        v = comm_ref[h, 1]
        qh = q_all[:, h * DH:(h + 1) * DH]
        s = lax.dot_general(
            qh, k, (((1,), (1,)), ((), ())),
            preferred_element_type=jnp.float32,
        ) * SCALE
        w = jnp.where(mask, jnp.exp(s), 0.0)
        p = (w / jnp.sum(w, axis=1, keepdims=True)).astype(jnp.bfloat16)
        ctx = jnp.dot(p, v, preferred_element_type=jnp.float32)
        ctx_ref[:, h * DH:(h + 1) * DH] = ctx.astype(jnp.bfloat16)

    out_ref[...] = jnp.dot(ctx_ref[...], wo_ref[...],
                           preferred_element_type=jnp.float32)

    for h in range(HQ):
        non_sender = 3 if h < HQ // 2 else 1

        @pl.when(my != non_sender)
        def _(h=h):
            snd[h].wait_send()


def kernel(x, Wq, K_ext, V_ext, Wo):
    bf16 = jnp.bfloat16
    xb = x[0].astype(bf16)
    wqb = Wq.astype(bf16)
    wob = Wo.astype(bf16)
    kvb = jnp.stack(
        [K_ext[0].astype(bf16).transpose(1, 0, 2),
         V_ext[0].astype(bf16).transpose(1, 0, 2)],
        axis=1,
    )

    out = pl.pallas_call(
        _body,
        out_shape=jax.ShapeDtypeStruct((SQ, D), jnp.float32),
        in_specs=[pl.BlockSpec(memory_space=pltpu.VMEM)] * 4,
        out_specs=pl.BlockSpec(memory_space=pltpu.VMEM),
        scratch_shapes=[
            pltpu.VMEM((HQ, 2, SKV, DH), bf16),
            pltpu.VMEM((SQ, D), bf16),
            pltpu.SemaphoreType.DMA((HQ,)),
            pltpu.SemaphoreType.DMA((HQ,)),
        ],
    )(xb, wqb, kvb, wob)

    return out.reshape(1, SQ, D)


# device time: 77305 ns/iter; 2.0841x vs baseline; 2.0841x over previous
import jax
import jax.numpy as jnp
from jax import lax
from jax.experimental import pallas as pl
from jax.experimental.pallas import tpu as pltpu

N_DEV = 4
SQ = 1024
SKV = 1024
HQ = 8
DH = 128
D = HQ * DH
BLK = 64
SCALE = 0.08838834764831843

ORDER_01 = (0, 1, 2, 3, 4, 5, 6, 7)
ORDER_3 = (4, 5, 6, 7, 0, 1, 2, 3)
ORDER_2 = (0, 4, 1, 5, 2, 6, 3, 7)


def _body(x_ref, wq_ref, kv_ref, wo_ref, out_ref,
          comm_ref, ctx_ref, send_sems, recv_sems):
    my = lax.axis_index("i")

    def mk(h, slot, tgt):
        return pltpu.make_async_remote_copy(
            src_ref=comm_ref.at[h],
            dst_ref=comm_ref.at[h],
            send_sem=send_sems.at[slot, h],
            recv_sem=recv_sems.at[h],
            device_id=(tgt,),
            device_id_type=pl.DeviceIdType.MESH,
        )

    to1 = [mk(h, 0, 1) for h in range(HQ)]
    to3 = [mk(h, 1, 3) for h in range(HQ)]
    rel = [mk(h, 0, 2) for h in range(HQ)]

    @pl.when(my == 0)
    def _():
        comm_ref[...] = kv_ref[...]
        for h in ORDER_01:
            to1[h].start()
        for h in ORDER_3:
            to3[h].start()

    xb = x_ref[0].astype(jnp.bfloat16)
    wqb = wq_ref[...].astype(jnp.bfloat16)
    q_all = jnp.dot(xb, wqb,
                    preferred_element_type=jnp.float32).astype(jnp.bfloat16)

    qb = lax.broadcasted_iota(jnp.int32, (SQ, SKV), 0) // BLK
    kb = lax.broadcasted_iota(jnp.int32, (SQ, SKV), 1) // BLK
    mask = kb <= qb

    def compute_head(h):
        k = comm_ref[h, 0]
        v = comm_ref[h, 1]
        qh = q_all[:, h * DH:(h + 1) * DH]
        s = lax.dot_general(
            qh, k, (((1,), (1,)), ((), ())),
            preferred_element_type=jnp.float32,
        ) * SCALE
        w = jnp.where(mask, jnp.exp(s), 0.0)
        p = (w / jnp.sum(w, axis=1, keepdims=True)).astype(jnp.bfloat16)
        ctx = jnp.dot(p, v, preferred_element_type=jnp.float32)
        ctx_ref[:, h * DH:(h + 1) * DH] = ctx.astype(jnp.bfloat16)

    @pl.when(my < 2)
    def _():
        for h in ORDER_01:
            @pl.when(my == 1)
            def _(h=h):
                to1[h].wait_recv()
                if h < HQ // 2:
                    rel[h].start()
            compute_head(h)

    @pl.when(my == 3)
    def _():
        for h in ORDER_3:
            to3[h].wait_recv()
            if h >= HQ // 2:
                rel[h].start()
            compute_head(h)

    @pl.when(my == 2)
    def _():
        for h in ORDER_2:
            rel[h].wait_recv()
            compute_head(h)

    wob = wo_ref[...].astype(jnp.bfloat16)
    out_ref[...] = jnp.dot(ctx_ref[...], wob,
                           preferred_element_type=jnp.float32)

    @pl.when(my == 0)
    def _():
        for h in range(HQ):
            to1[h].wait_send()
            to3[h].wait_send()

    @pl.when(my == 1)
    def _():
        for h in range(HQ // 2):
            rel[h].wait_send()

    @pl.when(my == 3)
    def _():
        for h in range(HQ // 2, HQ):
            rel[h].wait_send()


def kernel(x, Wq, K_ext, V_ext, Wo):
    bf16 = jnp.bfloat16
    kvb = jnp.stack(
        [K_ext[0].astype(bf16).transpose(1, 0, 2),
         V_ext[0].astype(bf16).transpose(1, 0, 2)],
        axis=1,
    )

    out = pl.pallas_call(
        _body,
        out_shape=jax.ShapeDtypeStruct((SQ, D), jnp.float32),
        in_specs=[pl.BlockSpec(memory_space=pltpu.VMEM)] * 4,
        out_specs=pl.BlockSpec(memory_space=pltpu.VMEM),
        scratch_shapes=[
            pltpu.VMEM((HQ, 2, SKV, DH), bf16),
            pltpu.VMEM((SQ, D), bf16),
            pltpu.SemaphoreType.DMA((2, HQ)),
            pltpu.SemaphoreType.DMA((HQ,)),
        ],
    )(x, Wq, kvb, Wo)

    return out.reshape(1, SQ, D)


# device time: 74366 ns/iter; 2.1665x vs baseline; 1.0395x over previous
import jax
import jax.numpy as jnp
from jax import lax
from jax.experimental import pallas as pl
from jax.experimental.pallas import tpu as pltpu

N_DEV = 4
SQ = 1024
SKV = 1024
HQ = 8
DH = 128
D = HQ * DH
BLK = 64
SCALE = 0.08838834764831843

ORDER_01 = (0, 1, 2, 3, 4, 5, 6, 7)
ORDER_3 = (4, 5, 6, 7, 0, 1, 2, 3)
ORDER_2 = (0, 4, 1, 5, 2, 6, 3, 7)


def _body(x_ref, wq_ref, kv_ref, wo_ref, out_ref,
          comm_ref, ctx_ref, send_sems, recv_sems):
    my = lax.axis_index("i")

    def mk(h, slot, tgt):
        return pltpu.make_async_remote_copy(
            src_ref=comm_ref.at[h],
            dst_ref=comm_ref.at[h],
            send_sem=send_sems.at[slot, h],
            recv_sem=recv_sems.at[h],
            device_id=(tgt,),
            device_id_type=pl.DeviceIdType.MESH,
        )

    to1 = [mk(h, 0, 1) for h in range(HQ)]
    to3 = [mk(h, 1, 3) for h in range(HQ)]
    rel = [mk(h, 0, 2) for h in range(HQ)]

    @pl.when(my == 0)
    def _():
        comm_ref[...] = kv_ref[...]
        for h in ORDER_01:
            to1[h].start()
        for h in ORDER_3:
            to3[h].start()

    xb = x_ref[0].astype(jnp.bfloat16)
    wqb = wq_ref[...].astype(jnp.bfloat16)
    q_all = jnp.dot(xb, wqb,
                    preferred_element_type=jnp.float32).astype(jnp.bfloat16)

    HALF = SQ // 2

    def blk_mask(q0, rows, cols):
        qb = (q0 + lax.broadcasted_iota(jnp.int32, (rows, cols), 0)) // BLK
        kb = lax.broadcasted_iota(jnp.int32, (rows, cols), 1) // BLK
        return kb <= qb

    mask_lo = blk_mask(0, HALF, HALF)
    mask_hi = blk_mask(HALF, HALF, SKV)

    def attend(qpart, k, v, mask):
        s = lax.dot_general(
            qpart, k, (((1,), (1,)), ((), ())),
            preferred_element_type=jnp.float32,
        ) * SCALE
        w = jnp.where(mask, jnp.exp(s), 0.0)
        p = (w / jnp.sum(w, axis=1, keepdims=True)).astype(jnp.bfloat16)
        return jnp.dot(p, v, preferred_element_type=jnp.float32)

    def compute_head(h):
        k = comm_ref[h, 0]
        v = comm_ref[h, 1]
        qh = q_all[:, h * DH:(h + 1) * DH]
        ctx_lo = attend(qh[:HALF], k[:HALF], v[:HALF], mask_lo)
        ctx_hi = attend(qh[HALF:], k, v, mask_hi)
        ctx_ref[:HALF, h * DH:(h + 1) * DH] = ctx_lo.astype(jnp.bfloat16)
        ctx_ref[HALF:, h * DH:(h + 1) * DH] = ctx_hi.astype(jnp.bfloat16)

    @pl.when(my < 2)
    def _():
        for h in ORDER_01:
            @pl.when(my == 1)
            def _(h=h):
                to1[h].wait_recv()
                if h < HQ // 2:
                    rel[h].start()
            compute_head(h)

    @pl.when(my == 3)
    def _():
        for h in ORDER_3:
            to3[h].wait_recv()
            if h >= HQ // 2:
                rel[h].start()
            compute_head(h)

    @pl.when(my == 2)
    def _():
        for h in ORDER_2:
            rel[h].wait_recv()
            compute_head(h)

    wob = wo_ref[...].astype(jnp.bfloat16)
    out_ref[...] = jnp.dot(ctx_ref[...], wob,
                           preferred_element_type=jnp.float32)

    @pl.when(my == 0)
    def _():
        for h in range(HQ):
            to1[h].wait_send()
            to3[h].wait_send()

    @pl.when(my == 1)
    def _():
        for h in range(HQ // 2):
            rel[h].wait_send()

    @pl.when(my == 3)
    def _():
        for h in range(HQ // 2, HQ):
            rel[h].wait_send()


def kernel(x, Wq, K_ext, V_ext, Wo):
    bf16 = jnp.bfloat16
    kvb = jnp.stack(
        [K_ext[0].astype(bf16).transpose(1, 0, 2),
         V_ext[0].astype(bf16).transpose(1, 0, 2)],
        axis=1,
    )

    out = pl.pallas_call(
        _body,
        out_shape=jax.ShapeDtypeStruct((SQ, D), jnp.float32),
        in_specs=[pl.BlockSpec(memory_space=pltpu.VMEM)] * 4,
        out_specs=pl.BlockSpec(memory_space=pltpu.VMEM),
        scratch_shapes=[
            pltpu.VMEM((HQ, 2, SKV, DH), bf16),
            pltpu.VMEM((SQ, D), bf16),
            pltpu.SemaphoreType.DMA((2, HQ)),
            pltpu.SemaphoreType.DMA((HQ,)),
        ],
    )(x, Wq, kvb, Wo)

    return out.reshape(1, SQ, D)
